# Initial kernel scaffold; baseline (speedup 1.0000x reference)
import jax
import jax.numpy as jnp
from jax.experimental import pallas as pl


def kernel(input_ids, attention_mask, embed_table, inv_freq):
    raise NotImplementedError("write your pallas kernel here")


# speedup vs baseline: 1.4453x; 1.4453x over previous
import functools

import jax
import jax.numpy as jnp
from jax import lax
from jax.experimental import pallas as pl
from jax.experimental.pallas import tpu as pltpu
from jax.experimental.pallas import tpu_sc as plsc

B = 2
S = 4096
D_MODEL = 1024
HEAD_DIM = 128
HALF = HEAD_DIM // 2

NC = 2
NS = 16
NW = NC * NS
ROWS = B * S
RPW = ROWS // NW
CHUNK = 64
NCHUNK = RPW // CHUNK

MASK_BS = 256
NEG = float(jnp.finfo(jnp.float32).min)

_sc_mesh = plsc.VectorSubcoreMesh(core_axis_name="c", subcore_axis_name="s")


@functools.partial(
    pl.kernel,
    mesh=_sc_mesh,
    out_type=jax.ShapeDtypeStruct((ROWS, D_MODEL), jnp.float32),
    scratch_types=[
        pltpu.VMEM((CHUNK,), jnp.int32),
        pltpu.VMEM((CHUNK, D_MODEL), jnp.float32),
        pltpu.SemaphoreType.DMA,
    ],
)
def _sc_gather(ids_hbm, table_hbm, out_hbm, idx_v, rows_v, sem):
    wid = lax.axis_index("s") * NC + lax.axis_index("c")
    base = wid * RPW
    for c in range(NCHUNK):
        off = base + c * CHUNK
        pltpu.sync_copy(ids_hbm.at[pl.ds(off, CHUNK)], idx_v)
        pltpu.async_copy(table_hbm.at[idx_v], rows_v, sem).wait()
        pltpu.sync_copy(rows_v, out_hbm.at[pl.ds(off, CHUNK)])


def _mask_body(am_ref, out_ref):
    i = pl.program_id(1)
    rows = lax.broadcasted_iota(jnp.int32, (MASK_BS, S), 0) + i * MASK_BS
    cols = lax.broadcasted_iota(jnp.int32, (MASK_BS, S), 1)
    pad = (am_ref[0, 0, :] == 0)[None, :]
    m = (cols > rows) | pad
    out_ref[0] = jnp.where(m, NEG, 0.0).astype(jnp.float32)


def _rope_body(invf_ref, cos_ref, sin_ref, pos_ref, cache_ref):
    pos = lax.broadcasted_iota(jnp.int32, (S, HALF), 0).astype(jnp.float32)
    freqs = pos * invf_ref[0, :][None, :]
    c = jnp.cos(freqs)
    s = jnp.sin(freqs)
    cos_ref[0] = jnp.concatenate([c, c], axis=-1)
    sin_ref[0] = jnp.concatenate([s, s], axis=-1)
    ids = lax.broadcasted_iota(jnp.int32, (1, S), 1)
    pos_ref[...] = ids
    cache_ref[...] = ids


def kernel(input_ids, attention_mask, embed_table, inv_freq):
    ids_flat = input_ids.reshape(ROWS)
    hidden = _sc_gather(ids_flat, embed_table).reshape(B, S, D_MODEL)

    mask3 = pl.pallas_call(
        _mask_body,
        grid=(B, S // MASK_BS),
        in_specs=[pl.BlockSpec((1, 1, S), lambda b, i: (b, 0, 0))],
        out_specs=pl.BlockSpec((1, MASK_BS, S), lambda b, i: (b, i, 0)),
        out_shape=jax.ShapeDtypeStruct((B, S, S), jnp.float32),
    )(attention_mask.reshape(B, 1, S))
    causal_mask = mask3[:, None, :, :]

    cos, sin, pos, cache = pl.pallas_call(
        _rope_body,
        out_shape=[
            jax.ShapeDtypeStruct((1, S, HEAD_DIM), jnp.float32),
            jax.ShapeDtypeStruct((1, S, HEAD_DIM), jnp.float32),
            jax.ShapeDtypeStruct((1, S), jnp.int32),
            jax.ShapeDtypeStruct((1, S), jnp.int32),
        ],
    )(inv_freq.reshape(1, HALF))

    return (hidden, causal_mask, pos, cache.reshape(S), cos, sin)


# speedup vs baseline: 1.5456x; 1.0694x over previous
import functools

import jax
import jax.numpy as jnp
from jax import lax
from jax.experimental import pallas as pl
from jax.experimental.pallas import tpu as pltpu
from jax.experimental.pallas import tpu_sc as plsc

B = 2
S = 4096
D_MODEL = 1024
HEAD_DIM = 128
HALF = HEAD_DIM // 2

NC = 2
NS = 16
NW = NC * NS
ROWS = B * S
RPW = ROWS // NW
CHUNK = 64
NCHUNK = RPW // CHUNK

MASK_BS = 512
NEG = float(jnp.finfo(jnp.float32).min)

_sc_mesh = plsc.VectorSubcoreMesh(core_axis_name="c", subcore_axis_name="s")


@functools.partial(
    pl.kernel,
    mesh=_sc_mesh,
    out_type=jax.ShapeDtypeStruct((ROWS, D_MODEL), jnp.float32),
    scratch_types=[
        pltpu.VMEM((CHUNK,), jnp.int32),
        pltpu.VMEM((CHUNK, D_MODEL), jnp.float32),
        pltpu.SemaphoreType.DMA,
    ],
)
def _sc_gather(ids_hbm, table_hbm, out_hbm, idx_v, rows_v, sem):
    wid = lax.axis_index("s") * NC + lax.axis_index("c")
    base = wid * RPW
    for c in range(NCHUNK):
        off = base + c * CHUNK
        pltpu.sync_copy(ids_hbm.at[pl.ds(off, CHUNK)], idx_v)
        pltpu.async_copy(table_hbm.at[idx_v], rows_v, sem).wait()
        pltpu.sync_copy(rows_v, out_hbm.at[pl.ds(off, CHUNK)])


def _mask_body(am_ref, out_ref):
    i = pl.program_id(1)
    rows = lax.broadcasted_iota(jnp.int32, (MASK_BS, S), 0) + i * MASK_BS
    cols = lax.broadcasted_iota(jnp.int32, (MASK_BS, S), 1)
    pad = (am_ref[0, 0, :] == 0)[None, :]
    m = (cols > rows) | pad
    out_ref[0] = jnp.where(m, NEG, 0.0).astype(jnp.float32)


def _rope_body(invf2_ref, cos_ref, sin_ref, pos_ref, cache_ref):
    NQ = S // 64
    w = invf2_ref[0, :][None, :]
    q_ang = (lax.broadcasted_iota(jnp.int32, (NQ, HEAD_DIM), 0) * 64).astype(jnp.float32) * w
    r_ang = lax.broadcasted_iota(jnp.int32, (64, HEAD_DIM), 0).astype(jnp.float32) * w
    cq = jnp.cos(q_ang)[:, None, :]
    sq = jnp.sin(q_ang)[:, None, :]
    cr = jnp.cos(r_ang)[None, :, :]
    sr = jnp.sin(r_ang)[None, :, :]
    cos_ref[0] = (cq * cr - sq * sr).reshape(S, HEAD_DIM)
    sin_ref[0] = (sq * cr + cq * sr).reshape(S, HEAD_DIM)
    ids = lax.broadcasted_iota(jnp.int32, (1, S), 1)
    pos_ref[...] = ids
    cache_ref[...] = ids


def kernel(input_ids, attention_mask, embed_table, inv_freq):
    ids_flat = input_ids.reshape(ROWS)
    hidden = _sc_gather(ids_flat, embed_table).reshape(B, S, D_MODEL)

    mask3 = pl.pallas_call(
        _mask_body,
        grid=(B, S // MASK_BS),
        in_specs=[pl.BlockSpec((1, 1, S), lambda b, i: (b, 0, 0))],
        out_specs=pl.BlockSpec((1, MASK_BS, S), lambda b, i: (b, i, 0)),
        out_shape=jax.ShapeDtypeStruct((B, S, S), jnp.float32),
    )(attention_mask.reshape(B, 1, S))
    causal_mask = mask3[:, None, :, :]

    cos, sin, pos, cache = pl.pallas_call(
        _rope_body,
        out_shape=[
            jax.ShapeDtypeStruct((1, S, HEAD_DIM), jnp.float32),
            jax.ShapeDtypeStruct((1, S, HEAD_DIM), jnp.float32),
            jax.ShapeDtypeStruct((1, S), jnp.int32),
            jax.ShapeDtypeStruct((1, S), jnp.int32),
        ],
    )(jnp.concatenate([inv_freq, inv_freq]).reshape(1, HEAD_DIM))

    return (hidden, causal_mask, pos, cache.reshape(S), cos, sin)
